# baseline (speedup 1.0000x reference)
import functools

import jax
import jax.numpy as jnp
from jax import lax
from jax.experimental import pallas as pl
from jax.experimental.pallas import tpu as pltpu
from jax.experimental.pallas import tpu_sc as plsc

MAX_LEN = 8192
EMBED_DIM = 1024

NUM_CORES = 2
NUM_SUBCORES = 16
NUM_WORKERS = NUM_CORES * NUM_SUBCORES

ROWS_PER_WORKER = MAX_LEN // NUM_WORKERS
CHUNK_SIZES = (8, 24, 40, 40, 40, 40, 40, 16, 8)
CHUNK_OFFS = tuple(sum(CHUNK_SIZES[:i]) for i in range(len(CHUNK_SIZES)))
assert sum(CHUNK_SIZES) == ROWS_PER_WORKER
MAX_CHUNK = max(CHUNK_SIZES)
NBUF = 3
LOOKAHEAD = 2
NUM_CHUNKS = len(CHUNK_SIZES)


@functools.partial(
    pl.kernel,
    mesh=plsc.VectorSubcoreMesh(core_axis_name="c", subcore_axis_name="s"),
    out_type=jax.ShapeDtypeStruct((MAX_LEN, EMBED_DIM), jnp.float32),
    scratch_types=(
        [pltpu.VMEM((ROWS_PER_WORKER,), jnp.int32)]
        + [pltpu.VMEM((MAX_CHUNK, EMBED_DIM), jnp.float32)] * NBUF
        + [pltpu.SemaphoreType.DMA] * (2 * NBUF)
    ),
)
def _sc_row_gather(idx_hbm, table_hbm, out_hbm, idx_v, *rest):
    bufs = rest[:NBUF]
    g_sems = rest[NBUF:2 * NBUF]
    s_sems = rest[2 * NBUF:]
    wid = lax.axis_index("s") * NUM_CORES + lax.axis_index("c")
    base = wid * ROWS_PER_WORKER

    def gather(c, b):
        sz = CHUNK_SIZES[c]
        return pltpu.async_copy(
            table_hbm.at[idx_v.at[pl.ds(CHUNK_OFFS[c], sz)]],
            bufs[b].at[pl.ds(0, sz)], g_sems[b])

    g_copies = [None] * NBUF
    s_copies = [None] * NBUF
    pltpu.sync_copy(idx_hbm.at[pl.ds(base, CHUNK_SIZES[0])],
                    idx_v.at[pl.ds(0, CHUNK_SIZES[0])])
    g_copies[0] = gather(0, 0)
    pltpu.sync_copy(
        idx_hbm.at[pl.ds(base + CHUNK_SIZES[0],
                         ROWS_PER_WORKER - CHUNK_SIZES[0])],
        idx_v.at[pl.ds(CHUNK_SIZES[0], ROWS_PER_WORKER - CHUNK_SIZES[0])])
    for c in range(1, min(LOOKAHEAD, NUM_CHUNKS)):
        g_copies[c % NBUF] = gather(c, c % NBUF)
    for j in range(NUM_CHUNKS):
        b = j % NBUF
        c = j + LOOKAHEAD
        if c < NUM_CHUNKS:
            bc = c % NBUF
            if c >= NBUF:
                s_copies[bc].wait()
            g_copies[bc] = gather(c, bc)
        g_copies[b].wait()
        s_copies[b] = pltpu.async_copy(
            bufs[b].at[pl.ds(0, CHUNK_SIZES[j])],
            out_hbm.at[pl.ds(base + CHUNK_OFFS[j], CHUNK_SIZES[j])],
            s_sems[b])
    for b in range(NBUF):
        if s_copies[b] is not None:
            s_copies[b].wait()


def kernel(seq_len, pe):
    shift = jnp.asarray(seq_len, jnp.int32) - MAX_LEN
    positions = jnp.clip(
        jnp.arange(MAX_LEN, dtype=jnp.int32) + shift, 0, MAX_LEN - 1)
    return _sc_row_gather(positions, pe)


# speedup vs baseline: 1.0156x; 1.0156x over previous
import functools

import jax
import jax.numpy as jnp
from jax import lax
from jax.experimental import pallas as pl
from jax.experimental.pallas import tpu as pltpu
from jax.experimental.pallas import tpu_sc as plsc

MAX_LEN = 8192
EMBED_DIM = 1024

NUM_CORES = 2
NUM_SUBCORES = 16
NUM_WORKERS = NUM_CORES * NUM_SUBCORES

ROWS_PER_WORKER = MAX_LEN // NUM_WORKERS
CHUNK_SIZES = (40, 40, 40, 40, 40, 40, 16)
CHUNK_OFFS = tuple(sum(CHUNK_SIZES[:i]) for i in range(len(CHUNK_SIZES)))
assert sum(CHUNK_SIZES) == ROWS_PER_WORKER
MAX_CHUNK = max(CHUNK_SIZES)
NBUF = 3
LOOKAHEAD = 2
NUM_CHUNKS = len(CHUNK_SIZES)


@functools.partial(
    pl.kernel,
    mesh=plsc.VectorSubcoreMesh(core_axis_name="c", subcore_axis_name="s"),
    out_type=jax.ShapeDtypeStruct((MAX_LEN, EMBED_DIM), jnp.float32),
    scratch_types=(
        [pltpu.VMEM((ROWS_PER_WORKER,), jnp.int32)]
        + [pltpu.VMEM((MAX_CHUNK, EMBED_DIM), jnp.float32)] * NBUF
        + [pltpu.SemaphoreType.DMA] * (2 * NBUF)
    ),
)
def _sc_row_gather(idx_hbm, table_hbm, out_hbm, idx_v, *rest):
    bufs = rest[:NBUF]
    g_sems = rest[NBUF:2 * NBUF]
    s_sems = rest[2 * NBUF:]
    wid = lax.axis_index("s") * NUM_CORES + lax.axis_index("c")
    base = wid * ROWS_PER_WORKER

    def gather(c, b):
        sz = CHUNK_SIZES[c]
        return pltpu.async_copy(
            table_hbm.at[idx_v.at[pl.ds(CHUNK_OFFS[c], sz)]],
            bufs[b].at[pl.ds(0, sz)], g_sems[b])

    g_copies = [None] * NBUF
    s_copies = [None] * NBUF
    pltpu.sync_copy(idx_hbm.at[pl.ds(base, CHUNK_SIZES[0])],
                    idx_v.at[pl.ds(0, CHUNK_SIZES[0])])
    g_copies[0] = gather(0, 0)
    pltpu.sync_copy(
        idx_hbm.at[pl.ds(base + CHUNK_SIZES[0],
                         ROWS_PER_WORKER - CHUNK_SIZES[0])],
        idx_v.at[pl.ds(CHUNK_SIZES[0], ROWS_PER_WORKER - CHUNK_SIZES[0])])
    for c in range(1, min(LOOKAHEAD, NUM_CHUNKS)):
        g_copies[c % NBUF] = gather(c, c % NBUF)
    for j in range(NUM_CHUNKS):
        b = j % NBUF
        c = j + LOOKAHEAD
        if c < NUM_CHUNKS:
            bc = c % NBUF
            if c >= NBUF:
                s_copies[bc].wait()
            g_copies[bc] = gather(c, bc)
        g_copies[b].wait()
        s_copies[b] = pltpu.async_copy(
            bufs[b].at[pl.ds(0, CHUNK_SIZES[j])],
            out_hbm.at[pl.ds(base + CHUNK_OFFS[j], CHUNK_SIZES[j])],
            s_sems[b])
    for b in range(NBUF):
        if s_copies[b] is not None:
            s_copies[b].wait()


def kernel(seq_len, pe):
    shift = jnp.asarray(seq_len, jnp.int32) - MAX_LEN
    positions = jnp.clip(
        jnp.arange(MAX_LEN, dtype=jnp.int32) + shift, 0, MAX_LEN - 1)
    return _sc_row_gather(positions, pe)
